# baseline (speedup 1.0000x reference)
import jax
import jax.numpy as jnp
from jax import lax
from jax.experimental import pallas as pl
from jax.experimental.pallas import tpu as pltpu
from jax.experimental.pallas import tpu_sc as plsc

N_ITEMS = 1000000
F = 16
B = 4096
N_DOCS = 100
HIST = 200
NDP = 112
HP = 208
NW = 32
ROWS_PER_W = B // NW


def _full16(v, dtype=jnp.int32):
  return jnp.full((16,), v, dtype=dtype)


_GATHER_DNUMS = lax.GatherDimensionNumbers(
    offset_dims=(), collapsed_slice_dims=(0,), start_index_map=(0,))


def _shuffle(v, perm):
  return lax.gather(v, perm[:, None], _GATHER_DNUMS, (1,),
                    mode=lax.GatherScatterMode.PROMISE_IN_BOUNDS)


def _lanesum(v):
  base = jnp.arange(16, dtype=jnp.int32)
  for k in (1, 2, 4, 8):
    v = v + _shuffle(v, base ^ k)
  return v


def _build_kernel():
  mesh = plsc.VectorSubcoreMesh(core_axis_name="c", subcore_axis_name="s")

  def body(iidx_hbm, pidx_hbm, coeff_hbm, itbl, btbl, ptbl, out_hbm,
           pidx_a, pidx_b, pidx_f, iidx_v, coeff_v,
           prows_v, irows_v, brow_v, prof_v, orow_v,
           pt0_v, it0_v, sidx, sg, sout):
    wid = lax.axis_index("s") * 2 + lax.axis_index("c")
    base = wid * ROWS_PER_W
    pltpu.sync_copy(ptbl.at[0], pt0_v)
    pltpu.sync_copy(itbl.at[0], it0_v)

    def idx_copies(b, g):
      return (
          pltpu.make_async_copy(pidx_hbm.at[pl.ds(g * HP, 104)],
                                pidx_a[b], sidx[b]),
          pltpu.make_async_copy(pidx_hbm.at[pl.ds(g * HP + 104, 104)],
                                pidx_b[b], sidx[b]),
          pltpu.make_async_copy(pidx_hbm.at[pl.ds(g * HP, HP)],
                                pidx_f[b], sidx[b]),
          pltpu.make_async_copy(iidx_hbm.at[pl.ds(g * NDP, NDP)],
                                iidx_v[b], sidx[b]),
          pltpu.make_async_copy(coeff_hbm.at[pl.ds(g * F, F)],
                                coeff_v[b], sidx[b]),
      )

    def gather_copies(b):
      return (
          pltpu.make_async_copy(ptbl.at[pidx_a[b]],
                                prows_v.at[b, pl.ds(0, 104)], sg[b]),
          pltpu.make_async_copy(ptbl.at[pidx_b[b]],
                                prows_v.at[b, pl.ds(104, 104)], sg[b]),
          pltpu.make_async_copy(itbl.at[iidx_v[b]], irows_v.at[b], sg[b]),
          pltpu.make_async_copy(btbl.at[iidx_v[b]], brow_v[b], sg[b]),
      )

    def out_copy(b, g):
      return pltpu.make_async_copy(
          orow_v.at[b], out_hbm.at[pl.ds(g * NDP, NDP)], sout[b])

    def issue(copies):
      for c in copies:
        c.start()

    def wait(copies):
      for c in copies:
        c.wait()

    def compute(b):
      pt0 = pt0_v[...]
      it0 = it0_v[...]
      iota = lax.iota(jnp.int32, 16)
      acc = [jnp.zeros((16,), jnp.float32) for _ in range(4)]
      for h in range(HP):
        acc[h % 4] = acc[h % 4] + prows_v[b, h, :]
      zcnt = jnp.zeros((16,), jnp.float32)
      for c in range(HP // 16):
        iv = pidx_f[b][pl.ds(16 * c, 16)]
        zcnt = zcnt + jnp.where(iv == 0, 1.0, 0.0).astype(jnp.float32)
      n0 = _lanesum(zcnt)
      prof = ((acc[0] + acc[1]) + (acc[2] + acc[3]) - n0 * pt0) \
          * coeff_v[b][...]
      prof_v[pl.ds(16, 16)] = prof

      pb = [plsc.load_gather(prof_v, [_full16(16 + f)]) for f in range(F)]
      dot0 = _lanesum(prof * it0)
      for c in range(NDP // 16):
        row_ids = iota + (16 * c)
        dots = jnp.zeros((16,), jnp.float32)
        for f in range(F):
          col = plsc.load_gather(irows_v.at[b], [row_ids, _full16(f)])
          dots = dots + pb[f] * col
        iv = iidx_v[b][pl.ds(16 * c, 16)]
        z = jnp.where(iv == 0, 1.0, 0.0).astype(jnp.float32)
        res = dots - z * dot0 + brow_v[b][pl.ds(16 * c, 16)]
        orow_v[b, pl.ds(16 * c, 16)] = res

    issue(idx_copies(0, base))
    issue(idx_copies(1, base + 1))
    wait(idx_copies(0, base))
    issue(gather_copies(0))
    out_copy(0, base).start()
    out_copy(1, base + 1).start()

    @pl.loop(0, ROWS_PER_W - 2, step=2)
    def _pair(q):
      for b in (0, 1):
        g = base + q + b
        wait(idx_copies(1 - b, g + 1))
        issue(gather_copies(1 - b))
        wait(gather_copies(b))
        out_copy(b, g).wait()
        compute(b)
        out_copy(b, g).start()
        issue(idx_copies(b, g + 2))

    g = base + ROWS_PER_W - 2
    wait(idx_copies(1, g + 1))
    issue(gather_copies(1))
    wait(gather_copies(0))
    out_copy(0, g).wait()
    compute(0)
    out_copy(0, g).start()

    wait(gather_copies(1))
    out_copy(1, g + 1).wait()
    compute(1)
    out_copy(1, g + 1).start()

    out_copy(0, g).wait()
    out_copy(1, g + 1).wait()

  return pl.kernel(
      body,
      out_type=jax.ShapeDtypeStruct((B * NDP,), jnp.float32),
      mesh=mesh,
      compiler_params=pltpu.CompilerParams(
          use_tc_tiling_on_sc=False, needs_layout_passes=False),
      scratch_types=[
          [pltpu.VMEM((104,), jnp.int32) for _ in range(2)],
          [pltpu.VMEM((104,), jnp.int32) for _ in range(2)],
          [pltpu.VMEM((HP,), jnp.int32) for _ in range(2)],
          [pltpu.VMEM((NDP,), jnp.int32) for _ in range(2)],
          [pltpu.VMEM((16,), jnp.float32) for _ in range(2)],
          pltpu.VMEM((2, HP, F), jnp.float32),
          pltpu.VMEM((2, NDP, F), jnp.float32),
          [pltpu.VMEM((NDP,), jnp.float32) for _ in range(2)],
          pltpu.VMEM((32,), jnp.float32),
          pltpu.VMEM((2, NDP), jnp.float32),
          pltpu.VMEM((16,), jnp.float32),
          pltpu.VMEM((16,), jnp.float32),
          [pltpu.SemaphoreType.DMA for _ in range(2)],
          [pltpu.SemaphoreType.DMA for _ in range(2)],
          [pltpu.SemaphoreType.DMA for _ in range(2)],
      ],
  )


def kernel(item_lst, past_items, len_past, item_table, item_bias_table,
           past_item_table):
  iidx = jnp.pad(item_lst.astype(jnp.int32),
                 ((0, 0), (0, NDP - N_DOCS))).reshape(-1) * 8
  pidx = jnp.pad(past_items.astype(jnp.int32),
                 ((0, 0), (0, HP - HIST))).reshape(-1) * 8
  coeff16 = jnp.broadcast_to(
      jnp.power(len_past, -0.5)[:, None].astype(jnp.float32), (B, F)
  ).reshape(-1)
  btbl = jnp.pad(item_bias_table, ((0, 0), (0, 7))).reshape(-1)
  itbl8 = jnp.pad(item_table, ((0, 0), (0, 112))).reshape(8 * N_ITEMS, F)
  ptbl8 = jnp.pad(past_item_table, ((0, 0), (0, 112))).reshape(
      8 * N_ITEMS, F)
  out = _build_kernel()(iidx, pidx, coeff16, itbl8, btbl, ptbl8)
  return out.reshape(B, NDP)[:, :N_DOCS]


# speedup vs baseline: 1.6955x; 1.6955x over previous
import jax
import jax.numpy as jnp
from jax import lax
from jax.experimental import pallas as pl
from jax.experimental.pallas import tpu as pltpu
from jax.experimental.pallas import tpu_sc as plsc

N_ITEMS = 1000000
F = 16
B = 4096
N_DOCS = 100
HIST = 200
NDP = 112
HP = 208
NW = 32
ROWS_PER_W = B // NW


def _full16(v, dtype=jnp.int32):
  return jnp.full((16,), v, dtype=dtype)


_GATHER_DNUMS = lax.GatherDimensionNumbers(
    offset_dims=(), collapsed_slice_dims=(0,), start_index_map=(0,))


def _shuffle(v, perm):
  return lax.gather(v, perm[:, None], _GATHER_DNUMS, (1,),
                    mode=lax.GatherScatterMode.PROMISE_IN_BOUNDS)


def _lanesum(v):
  base = jnp.arange(16, dtype=jnp.int32)
  for k in (1, 2, 4, 8):
    v = v + _shuffle(v, base ^ k)
  return v


def _build_kernel():
  mesh = plsc.VectorSubcoreMesh(core_axis_name="c", subcore_axis_name="s")

  def body(iidx_hbm, pidx_hbm, coeff_hbm, itbl, btbl, ptbl, out_hbm,
           pidx_a, pidx_b, pidx_f, iidx_v, coeff_v,
           prows_v, irows_v, brow_v, prof_v, orow_v,
           pt0_v, it0_v, sidx, sg, sout):
    wid = lax.axis_index("s") * 2 + lax.axis_index("c")
    base = wid * ROWS_PER_W
    pltpu.sync_copy(ptbl.at[0], pt0_v)
    pltpu.sync_copy(itbl.at[0], it0_v)

    def idx_copies(b, g):
      return (
          pltpu.make_async_copy(pidx_hbm.at[pl.ds(g * HP, 104)],
                                pidx_a[b], sidx[b]),
          pltpu.make_async_copy(pidx_hbm.at[pl.ds(g * HP + 104, 104)],
                                pidx_b[b], sidx[b]),
          pltpu.make_async_copy(pidx_hbm.at[pl.ds(g * HP, HP)],
                                pidx_f[b], sidx[b]),
          pltpu.make_async_copy(iidx_hbm.at[pl.ds(g * NDP, NDP)],
                                iidx_v[b], sidx[b]),
          pltpu.make_async_copy(coeff_hbm.at[pl.ds(g * F, F)],
                                coeff_v[b], sidx[b]),
      )

    def gather_copies(b):
      return (
          pltpu.make_async_copy(ptbl.at[pidx_a[b]],
                                prows_v.at[b, pl.ds(0, 104)], sg[b]),
          pltpu.make_async_copy(ptbl.at[pidx_b[b]],
                                prows_v.at[b, pl.ds(104, 104)], sg[b]),
          pltpu.make_async_copy(itbl.at[iidx_v[b]], irows_v.at[b], sg[b]),
          pltpu.make_async_copy(btbl.at[iidx_v[b]], brow_v[b], sg[b]),
      )

    def out_copy(b, g):
      return pltpu.make_async_copy(
          orow_v.at[b], out_hbm.at[pl.ds(g * NDP, NDP)], sout[b])

    def issue(copies):
      for c in copies:
        c.start()

    def wait(copies):
      for c in copies:
        c.wait()

    def compute(b):
      pt0 = pt0_v[...]
      it0 = it0_v[...]
      iota = lax.iota(jnp.int32, 16)
      acc = [jnp.zeros((16,), jnp.float32) for _ in range(4)]
      for h in range(HP):
        acc[h % 4] = acc[h % 4] + prows_v[b, h, :]
      zcnt = jnp.zeros((16,), jnp.float32)
      for c in range(HP // 16):
        iv = pidx_f[b][pl.ds(16 * c, 16)]
        zcnt = zcnt + jnp.where(iv == 0, 1.0, 0.0).astype(jnp.float32)
      n0 = _lanesum(zcnt)
      prof = ((acc[0] + acc[1]) + (acc[2] + acc[3]) - n0 * pt0) \
          * coeff_v[b][...]
      prof_v[pl.ds(16, 16)] = prof

      pb = [plsc.load_gather(prof_v, [_full16(16 + f)]) for f in range(F)]
      dot0 = _lanesum(prof * it0)
      for c in range(NDP // 16):
        row_ids = iota + (16 * c)
        dots = jnp.zeros((16,), jnp.float32)
        for f in range(F):
          col = plsc.load_gather(irows_v.at[b], [row_ids, _full16(f)])
          dots = dots + pb[f] * col
        iv = iidx_v[b][pl.ds(16 * c, 16)]
        z = jnp.where(iv == 0, 1.0, 0.0).astype(jnp.float32)
        res = dots - z * dot0 + brow_v[b][pl.ds(16 * c, 16)]
        orow_v[b, pl.ds(16 * c, 16)] = res

    issue(idx_copies(0, base))
    issue(idx_copies(1, base + 1))
    wait(idx_copies(0, base))
    issue(gather_copies(0))
    out_copy(0, base).start()
    out_copy(1, base + 1).start()

    @pl.loop(0, ROWS_PER_W - 2, step=2)
    def _pair(q):
      for b in (0, 1):
        g = base + q + b
        wait(idx_copies(1 - b, g + 1))
        issue(gather_copies(1 - b))
        wait(gather_copies(b))
        out_copy(b, g).wait()
        compute(b)
        out_copy(b, g).start()
        issue(idx_copies(b, g + 2))

    g = base + ROWS_PER_W - 2
    wait(idx_copies(1, g + 1))
    issue(gather_copies(1))
    wait(gather_copies(0))
    out_copy(0, g).wait()
    compute(0)
    out_copy(0, g).start()

    wait(gather_copies(1))
    out_copy(1, g + 1).wait()
    compute(1)
    out_copy(1, g + 1).start()

    out_copy(0, g).wait()
    out_copy(1, g + 1).wait()

  return pl.kernel(
      body,
      out_type=jax.ShapeDtypeStruct((B * NDP,), jnp.float32),
      mesh=mesh,
      compiler_params=pltpu.CompilerParams(
          use_tc_tiling_on_sc=False, needs_layout_passes=False),
      scratch_types=[
          [pltpu.VMEM((104,), jnp.int32) for _ in range(2)],
          [pltpu.VMEM((104,), jnp.int32) for _ in range(2)],
          [pltpu.VMEM((HP,), jnp.int32) for _ in range(2)],
          [pltpu.VMEM((NDP,), jnp.int32) for _ in range(2)],
          [pltpu.VMEM((16,), jnp.float32) for _ in range(2)],
          pltpu.VMEM((2, HP, F), jnp.float32),
          pltpu.VMEM((2, NDP, F), jnp.float32),
          [pltpu.VMEM((NDP,), jnp.float32) for _ in range(2)],
          pltpu.VMEM((32,), jnp.float32),
          pltpu.VMEM((2, NDP), jnp.float32),
          pltpu.VMEM((16,), jnp.float32),
          pltpu.VMEM((16,), jnp.float32),
          [pltpu.SemaphoreType.DMA for _ in range(2)],
          [pltpu.SemaphoreType.DMA for _ in range(2)],
          [pltpu.SemaphoreType.DMA for _ in range(2)],
      ],
  )


def kernel(item_lst, past_items, len_past, item_table, item_bias_table,
           past_item_table):
  iidx = jnp.pad(item_lst.astype(jnp.int32),
                 ((0, 0), (0, NDP - N_DOCS))).reshape(-1)
  pidx = jnp.pad(past_items.astype(jnp.int32),
                 ((0, 0), (0, HP - HIST))).reshape(-1)
  coeff16 = jnp.broadcast_to(
      jnp.power(len_past, -0.5)[:, None].astype(jnp.float32), (B, F)
  ).reshape(-1)
  btbl = item_bias_table[:, 0]
  out = _build_kernel()(iidx, pidx, coeff16, item_table, btbl,
                        past_item_table)
  return out.reshape(B, NDP)[:, :N_DOCS]


# speedup vs baseline: 1.6986x; 1.0019x over previous
import jax
import jax.numpy as jnp
from jax import lax
from jax.experimental import pallas as pl
from jax.experimental.pallas import tpu as pltpu
from jax.experimental.pallas import tpu_sc as plsc

N_ITEMS = 1000000
F = 16
B = 4096
N_DOCS = 100
HIST = 200
NDP = 112
HP = 208
NW = 32
ROWS_PER_W = B // NW


def _full16(v, dtype=jnp.int32):
  return jnp.full((16,), v, dtype=dtype)


_GATHER_DNUMS = lax.GatherDimensionNumbers(
    offset_dims=(), collapsed_slice_dims=(0,), start_index_map=(0,))


def _shuffle(v, perm):
  return lax.gather(v, perm[:, None], _GATHER_DNUMS, (1,),
                    mode=lax.GatherScatterMode.PROMISE_IN_BOUNDS)


def _lanesum(v):
  base = jnp.arange(16, dtype=jnp.int32)
  for k in (1, 2, 4, 8):
    v = v + _shuffle(v, base ^ k)
  return v


def _build_kernel():
  mesh = plsc.VectorSubcoreMesh(core_axis_name="c", subcore_axis_name="s")

  def body(iidx_hbm, pidx_hbm, coeff_hbm, itbl, btbl, ptbl, out_hbm,
           pidx_f, iidx_v, coeff_v,
           prows_v, irows_v, brow_v, prof_v, orow_v,
           pt0_v, it0_v, sidx, sg, sout):
    wid = lax.axis_index("s") * 2 + lax.axis_index("c")
    base = wid * ROWS_PER_W
    pltpu.sync_copy(ptbl.at[0], pt0_v)
    pltpu.sync_copy(itbl.at[0], it0_v)

    def idx_copies(b, g):
      return (
          pltpu.make_async_copy(pidx_hbm.at[pl.ds(g * HP, HP)],
                                pidx_f[b], sidx[b]),
          pltpu.make_async_copy(iidx_hbm.at[pl.ds(g * NDP, NDP)],
                                iidx_v[b], sidx[b]),
          pltpu.make_async_copy(coeff_hbm.at[pl.ds(g * F, F)],
                                coeff_v[b], sidx[b]),
      )

    def gather_copies(b):
      return (
          pltpu.make_async_copy(ptbl.at[pidx_f[b].at[pl.ds(0, 104)]],
                                prows_v.at[b, pl.ds(0, 104)], sg[b]),
          pltpu.make_async_copy(ptbl.at[pidx_f[b].at[pl.ds(104, 104)]],
                                prows_v.at[b, pl.ds(104, 104)], sg[b]),
          pltpu.make_async_copy(itbl.at[iidx_v[b]], irows_v.at[b], sg[b]),
          pltpu.make_async_copy(btbl.at[iidx_v[b]], brow_v[b], sg[b]),
      )

    def out_copy(b, g):
      return pltpu.make_async_copy(
          orow_v.at[b], out_hbm.at[pl.ds(g * NDP, NDP)], sout[b])

    def issue(copies):
      for c in copies:
        c.start()

    def wait(copies):
      for c in copies:
        c.wait()

    def compute(b):
      pt0 = pt0_v[...]
      it0 = it0_v[...]
      iota = lax.iota(jnp.int32, 16)
      acc = [jnp.zeros((16,), jnp.float32) for _ in range(4)]
      for h in range(HP):
        acc[h % 4] = acc[h % 4] + prows_v[b, h, :]
      zcnt = jnp.zeros((16,), jnp.float32)
      for c in range(HP // 16):
        iv = pidx_f[b][pl.ds(16 * c, 16)]
        zcnt = zcnt + jnp.where(iv == 0, 1.0, 0.0).astype(jnp.float32)
      n0 = _lanesum(zcnt)
      prof = ((acc[0] + acc[1]) + (acc[2] + acc[3]) - n0 * pt0) \
          * coeff_v[b][...]
      prof_v[pl.ds(16, 16)] = prof

      pb = [plsc.load_gather(prof_v, [_full16(16 + f)]) for f in range(F)]
      dot0 = _lanesum(prof * it0)
      for c in range(NDP // 16):
        row_ids = iota + (16 * c)
        dacc = [jnp.zeros((16,), jnp.float32) for _ in range(4)]
        for f in range(F):
          col = plsc.load_gather(irows_v.at[b], [row_ids, _full16(f)])
          dacc[f % 4] = dacc[f % 4] + pb[f] * col
        dots = (dacc[0] + dacc[1]) + (dacc[2] + dacc[3])
        iv = iidx_v[b][pl.ds(16 * c, 16)]
        z = jnp.where(iv == 0, 1.0, 0.0).astype(jnp.float32)
        res = dots - z * dot0 + brow_v[b][pl.ds(16 * c, 16)]
        orow_v[b, pl.ds(16 * c, 16)] = res

    issue(idx_copies(0, base))
    issue(idx_copies(1, base + 1))
    wait(idx_copies(0, base))
    issue(gather_copies(0))
    out_copy(0, base).start()
    out_copy(1, base + 1).start()

    @pl.loop(0, ROWS_PER_W - 2, step=2)
    def _pair(q):
      for b in (0, 1):
        g = base + q + b
        wait(idx_copies(1 - b, g + 1))
        issue(gather_copies(1 - b))
        wait(gather_copies(b))
        out_copy(b, g).wait()
        compute(b)
        out_copy(b, g).start()
        issue(idx_copies(b, g + 2))

    g = base + ROWS_PER_W - 2
    wait(idx_copies(1, g + 1))
    issue(gather_copies(1))
    wait(gather_copies(0))
    out_copy(0, g).wait()
    compute(0)
    out_copy(0, g).start()

    wait(gather_copies(1))
    out_copy(1, g + 1).wait()
    compute(1)
    out_copy(1, g + 1).start()

    out_copy(0, g).wait()
    out_copy(1, g + 1).wait()

  return pl.kernel(
      body,
      out_type=jax.ShapeDtypeStruct((B * NDP,), jnp.float32),
      mesh=mesh,
      compiler_params=pltpu.CompilerParams(
          use_tc_tiling_on_sc=False, needs_layout_passes=False),
      scratch_types=[
          [pltpu.VMEM((HP,), jnp.int32) for _ in range(2)],
          [pltpu.VMEM((NDP,), jnp.int32) for _ in range(2)],
          [pltpu.VMEM((16,), jnp.float32) for _ in range(2)],
          pltpu.VMEM((2, HP, F), jnp.float32),
          pltpu.VMEM((2, NDP, F), jnp.float32),
          [pltpu.VMEM((NDP,), jnp.float32) for _ in range(2)],
          pltpu.VMEM((32,), jnp.float32),
          pltpu.VMEM((2, NDP), jnp.float32),
          pltpu.VMEM((16,), jnp.float32),
          pltpu.VMEM((16,), jnp.float32),
          [pltpu.SemaphoreType.DMA for _ in range(2)],
          [pltpu.SemaphoreType.DMA for _ in range(2)],
          [pltpu.SemaphoreType.DMA for _ in range(2)],
      ],
  )


def kernel(item_lst, past_items, len_past, item_table, item_bias_table,
           past_item_table):
  iidx = jnp.pad(item_lst.astype(jnp.int32),
                 ((0, 0), (0, NDP - N_DOCS))).reshape(-1)
  pidx = jnp.pad(past_items.astype(jnp.int32),
                 ((0, 0), (0, HP - HIST))).reshape(-1)
  coeff16 = jnp.broadcast_to(
      jnp.power(len_past, -0.5)[:, None].astype(jnp.float32), (B, F)
  ).reshape(-1)
  btbl = item_bias_table[:, 0]
  out = _build_kernel()(iidx, pidx, coeff16, item_table, btbl,
                        past_item_table)
  return out.reshape(B, NDP)[:, :N_DOCS]
